# Initial kernel scaffold; baseline (speedup 1.0000x reference)
import jax
import jax.numpy as jnp
from jax.experimental import pallas as pl


def kernel(x, edge_index, row_node_idx, W_in, b_in, W1, b1, W2, b2, W3, b3, W4, b4, W5, b5):
    raise NotImplementedError("write your pallas kernel here")


# speedup vs baseline: 6.2774x; 6.2774x over previous
import functools

import jax
import jax.numpy as jnp
from jax import lax
from jax.experimental import pallas as pl
from jax.experimental.pallas import tpu as pltpu
from jax.experimental.pallas import tpu_sc as plsc

_NC, _NS, _L = 2, 16, 16
_NW = _NC * _NS

_N = 10000
_NP = 10240
_E = 320000
_H = 128

_MESH = plsc.VectorSubcoreMesh(
    core_axis_name="c", subcore_axis_name="s", num_cores=_NC, num_subcores=_NS
)


@functools.cache
def _deg_kernel(n_ch, ch):
    rows_pt = _NP // _NS

    def body(idx_hbm, ones_hbm, z_hbm, out_hbm, idx_v, buf, acc):
        c = lax.axis_index("c")
        s = lax.axis_index("s")
        w = c * _NS + s
        pltpu.sync_copy(idx_hbm.at[w], idx_v)
        pltpu.sync_copy(ones_hbm, buf)
        pltpu.sync_copy(z_hbm, acc.at[pl.ds(s * rows_pt, rows_pt)])
        plsc.subcore_barrier()

        def step(j, carry):
            pltpu.sync_copy(buf, acc.at[idx_v.at[j]], add=True)
            return carry

        lax.fori_loop(0, n_ch, step, 0)
        plsc.subcore_barrier()
        pltpu.sync_copy(
            acc.at[pl.ds(s * rows_pt, rows_pt)],
            out_hbm.at[pl.ds(c * _NP + s * rows_pt, rows_pt)],
        )

    return pl.kernel(
        body,
        out_type=jax.ShapeDtypeStruct((_NC * _NP, _H), jnp.float32),
        mesh=_MESH,
        scratch_types=[
            pltpu.VMEM((n_ch, ch), jnp.int32),
            pltpu.VMEM((ch, _H), jnp.float32),
            pltpu.VMEM_SHARED((_NP, _H), jnp.float32),
        ],
    )


@functools.cache
def _gs_kernel(n_ch, ch, acc_rows):
    rows_pt = acc_rows // _NS

    def body(table_hbm, gidx_hbm, sidx_hbm, zrow_hbm, out_hbm, gidx_v, sidx_v, buf, acc):
        c = lax.axis_index("c")
        s = lax.axis_index("s")
        w = c * _NS + s
        pltpu.sync_copy(gidx_hbm.at[w], gidx_v)
        pltpu.sync_copy(sidx_hbm.at[w], sidx_v)
        pltpu.sync_copy(zrow_hbm.at[pl.ds(0, rows_pt)], acc.at[pl.ds(s * rows_pt, rows_pt)])
        plsc.subcore_barrier()

        def step(j, carry):
            pltpu.sync_copy(table_hbm.at[gidx_v.at[j]], buf)
            pltpu.sync_copy(buf, acc.at[sidx_v.at[j]], add=True)
            return carry

        lax.fori_loop(0, n_ch, step, 0)
        plsc.subcore_barrier()
        pltpu.sync_copy(
            acc.at[pl.ds(s * rows_pt, rows_pt)],
            out_hbm.at[pl.ds(c * acc_rows + s * rows_pt, rows_pt)],
        )

    return pl.kernel(
        body,
        out_type=jax.ShapeDtypeStruct((_NC * acc_rows, _H), jnp.float32),
        mesh=_MESH,
        scratch_types=[
            pltpu.VMEM((n_ch, ch), jnp.int32),
            pltpu.VMEM((n_ch, ch), jnp.int32),
            pltpu.VMEM((ch, _H), jnp.float32),
            pltpu.VMEM_SHARED((acc_rows, _H), jnp.float32),
        ],
    )


_BLK = 1024


def _tc_input_linear(x, degt, w, b):
    def body(x_ref, deg_ref, w_ref, b_ref, o_ref):
        oisq = lax.rsqrt(jnp.clip(deg_ref[0][:, 0], 1.0, None))
        h = jnp.dot(x_ref[...], w_ref[...], preferred_element_type=jnp.float32)
        h = jnp.maximum(h + b_ref[...], 0.0)
        o_ref[...] = h * oisq[:, None]

    return pl.pallas_call(
        body,
        grid=(_NP // _BLK,),
        in_specs=[
            pl.BlockSpec((_BLK, _H), lambda i: (i, 0)),
            pl.BlockSpec((1, _BLK, _NC), lambda i: (i, 0, 0)),
            pl.BlockSpec((_H, _H), lambda i: (0, 0)),
            pl.BlockSpec((1, _H), lambda i: (0, 0)),
        ],
        out_specs=pl.BlockSpec((_BLK, _H), lambda i: (i, 0)),
        out_shape=jax.ShapeDtypeStruct((_NP, _H), jnp.float32),
    )(x, degt, w, b)


def _tc_gconv(aggp, degt, w, b, *, relu, oscale):
    def body(aggp_ref, deg_ref, w_ref, b_ref, o_ref):
        dd = deg_ref[0]
        iisq = lax.rsqrt(jnp.clip(dd[:, 1], 1.0, None))
        agg = (aggp_ref[0] + aggp_ref[1]) * iisq[:, None]
        h = jnp.dot(agg, w_ref[...], preferred_element_type=jnp.float32) + b_ref[...]
        if relu:
            h = jnp.maximum(h, 0.0)
        if oscale:
            oisq = lax.rsqrt(jnp.clip(dd[:, 0], 1.0, None))
            h = h * oisq[:, None]
        o_ref[...] = h

    return pl.pallas_call(
        body,
        grid=(_NP // _BLK,),
        in_specs=[
            pl.BlockSpec((_NC, _BLK, _H), lambda i: (0, i, 0)),
            pl.BlockSpec((1, _BLK, _NC), lambda i: (i, 0, 0)),
            pl.BlockSpec((_H, _H), lambda i: (0, 0)),
            pl.BlockSpec((1, _H), lambda i: (0, 0)),
        ],
        out_specs=pl.BlockSpec((_BLK, _H), lambda i: (i, 0)),
        out_shape=jax.ShapeDtypeStruct((_NP, _H), jnp.float32),
    )(aggp, degt, w, b)


def _tc_head(featsum, f_scale, w3, b3, w4, b4, w5p, b5p):
    r = featsum.shape[0]

    def body(f_ref, w3_ref, b3_ref, w4_ref, b4_ref, w5_ref, b5_ref, o_ref):
        z = f_ref[...] * f_scale
        z = jnp.maximum(jnp.dot(z, w3_ref[...], preferred_element_type=jnp.float32) + b3_ref[...], 0.0)
        z = jnp.maximum(jnp.dot(z, w4_ref[...], preferred_element_type=jnp.float32) + b4_ref[...], 0.0)
        o_ref[...] = jnp.dot(z, w5_ref[...], preferred_element_type=jnp.float32) + b5_ref[...]

    full = lambda i: (0, 0)
    return pl.pallas_call(
        body,
        grid=(r // _BLK,),
        in_specs=[
            pl.BlockSpec((_BLK, _H), lambda i: (i, 0)),
            pl.BlockSpec((_H, _H), full),
            pl.BlockSpec((1, _H), full),
            pl.BlockSpec((_H, _H), full),
            pl.BlockSpec((1, _H), full),
            pl.BlockSpec((_H, _H), full),
            pl.BlockSpec((1, _H), full),
        ],
        out_specs=pl.BlockSpec((_BLK, _H), lambda i: (i, 0)),
        out_shape=jax.ShapeDtypeStruct((r, _H), jnp.float32),
    )(featsum, w3, b3, w4, b4, w5p, b5p)


def kernel(x, edge_index, row_node_idx, W_in, b_in, W1, b1, W2, b2, W3, b3, W4, b4, W5, b5):
    r, f = row_node_idx.shape
    epw = _E // _NW
    e_ch = 100
    src3 = edge_index[0].reshape(_NW, epw // e_ch, e_ch)
    dst3 = edge_index[1].reshape(_NW, epw // e_ch, e_ch)

    epw_d = _E // _NS
    didx = jnp.stack(
        [edge_index[0].reshape(_NS, epw_d // e_ch, e_ch),
         edge_index[1].reshape(_NS, epw_d // e_ch, e_ch)]
    ).reshape(_NW, epw_d // e_ch, e_ch)

    rows_per_core = r // _NC
    rf_pw = r * f // _NW
    r_ch = 4 * f
    gidx_row = row_node_idx.reshape(_NW, rf_pw // r_ch, r_ch)
    sidx_row = (lax.broadcasted_iota(jnp.int32, (r, f), 0) % rows_per_core).reshape(
        _NW, rf_pw // r_ch, r_ch
    )

    ones128 = jnp.ones((e_ch, _H), jnp.float32)
    zrow = jnp.zeros((_NP // _NS, _H), jnp.float32)
    x_p = jnp.pad(x, ((0, _NP - _N), (0, 0)))

    dpo = _deg_kernel(epw_d // e_ch, e_ch)(didx, ones128, zrow)
    degt = dpo[:, 0].reshape(_NC, _NP).T.reshape(_NP // _BLK, _BLK, _NC)

    h0s = _tc_input_linear(x_p, degt, W_in, b_in.reshape(1, _H))

    gs_edge = _gs_kernel(epw // e_ch, e_ch, _NP)
    agg1 = gs_edge(h0s, src3, dst3, zrow).reshape(_NC, _NP, _H)
    h1s = _tc_gconv(agg1, degt, W1, b1.reshape(1, _H), relu=True, oscale=True)
    agg2 = gs_edge(h1s, src3, dst3, zrow).reshape(_NC, _NP, _H)
    h2 = _tc_gconv(agg2, degt, W2, b2.reshape(1, _H), relu=False, oscale=False)

    gs_row = _gs_kernel(rf_pw // r_ch, r_ch, rows_per_core)
    featsum = gs_row(h2, gidx_row, sidx_row, zrow)

    c = W5.shape[1]
    w5p = jnp.pad(W5, ((0, 0), (0, _H - c)))
    b5p = jnp.pad(b5, (0, _H - c)).reshape(1, _H)
    out = _tc_head(
        featsum, 1.0 / f, W3, b3.reshape(1, _H), W4, b4.reshape(1, _H), w5p, b5p
    )
    return out[:, :c]


# speedup vs baseline: 9.1095x; 1.4512x over previous
import functools

import jax
import jax.numpy as jnp
from jax import lax
from jax.experimental import pallas as pl
from jax.experimental.pallas import tpu as pltpu
from jax.experimental.pallas import tpu_sc as plsc

_NC, _NS, _L = 2, 16, 16
_NW = _NC * _NS

_N = 10000
_NP = 10240
_E = 320000
_H = 128

_MESH = plsc.VectorSubcoreMesh(
    core_axis_name="c", subcore_axis_name="s", num_cores=_NC, num_subcores=_NS
)


@functools.cache
def _deg_kernel(n_ch, ch):
    rows_pt = _NP // _NS

    k = 8

    def body(idx_hbm, ones_hbm, z_hbm, out_hbm, idx_v, buf, acc, sem):
        c = lax.axis_index("c")
        s = lax.axis_index("s")
        w = c * _NS + s
        pltpu.sync_copy(idx_hbm.at[w], idx_v)
        pltpu.sync_copy(ones_hbm, buf)
        pltpu.sync_copy(z_hbm, acc.at[pl.ds(s * rows_pt, rows_pt)])
        plsc.subcore_barrier()

        def group(t, carry):
            base = t * k
            descs = [
                pltpu.async_copy(buf, acc.at[idx_v.at[base + b]], sem, add=True)
                for b in range(k)
            ]
            for d in descs:
                d.wait()
            return carry

        lax.fori_loop(0, n_ch // k, group, 0)
        plsc.subcore_barrier()
        pltpu.sync_copy(
            acc.at[pl.ds(s * rows_pt, rows_pt)],
            out_hbm.at[pl.ds(c * _NP + s * rows_pt, rows_pt)],
        )

    return pl.kernel(
        body,
        out_type=jax.ShapeDtypeStruct((_NC * _NP, _H), jnp.float32),
        mesh=_MESH,
        scratch_types=[
            pltpu.VMEM((n_ch, ch), jnp.int32),
            pltpu.VMEM((ch, _H), jnp.float32),
            pltpu.VMEM_SHARED((_NP, _H), jnp.float32),
            pltpu.SemaphoreType.DMA,
        ],
    )


@functools.cache
def _gs_kernel(n_ch, ch, acc_rows, phases=1):
    rows_pt = acc_rows // _NS
    n_ch_p = n_ch // phases

    def body(table_hbm, idx_hbm, zrow_hbm, out_hbm, idx_v, bufd, acc, sem0, sem1):
        c = lax.axis_index("c")
        s = lax.axis_index("s")
        w = c * _NS + s
        pltpu.sync_copy(zrow_hbm.at[pl.ds(0, rows_pt)], acc.at[pl.ds(s * rows_pt, rows_pt)])
        bufs = (bufd.at[0], bufd.at[1])
        sems = (sem0, sem1)
        plsc.subcore_barrier()

        for p in range(phases):
            pltpu.sync_copy(idx_hbm.at[w].at[p], idx_v)
            gidx = idx_v.at[0]
            sidx = idx_v.at[1]
            pending = pltpu.async_copy(table_hbm.at[gidx.at[0]], bufs[0], sems[0])
            for j in range(n_ch_p):
                nxt = None
                if j + 1 < n_ch_p:
                    nxt = pltpu.async_copy(
                        table_hbm.at[gidx.at[j + 1]], bufs[(j + 1) % 2], sems[(j + 1) % 2]
                    )
                pending.wait()
                pltpu.sync_copy(bufs[j % 2], acc.at[sidx.at[j]], add=True)
                pending = nxt
        plsc.subcore_barrier()
        pltpu.sync_copy(
            acc.at[pl.ds(s * rows_pt, rows_pt)],
            out_hbm.at[pl.ds(c * acc_rows + s * rows_pt, rows_pt)],
        )

    return pl.kernel(
        body,
        out_type=jax.ShapeDtypeStruct((_NC * acc_rows, _H), jnp.float32),
        mesh=_MESH,
        scratch_types=[
            pltpu.VMEM((2, n_ch_p, ch), jnp.int32),
            pltpu.VMEM((2, ch, _H), jnp.float32),
            pltpu.VMEM_SHARED((acc_rows, _H), jnp.float32),
            pltpu.SemaphoreType.DMA,
            pltpu.SemaphoreType.DMA,
        ],
    )


_BLK = 1024


def _tc_input_linear(x, degt, w, b):
    def body(x_ref, deg_ref, w_ref, b_ref, o_ref):
        oisq = lax.rsqrt(jnp.clip(deg_ref[0][:, 0], 1.0, None))
        h = jnp.dot(x_ref[...], w_ref[...], preferred_element_type=jnp.float32)
        h = jnp.maximum(h + b_ref[...], 0.0)
        o_ref[...] = h * oisq[:, None]

    return pl.pallas_call(
        body,
        grid=(_NP // _BLK,),
        in_specs=[
            pl.BlockSpec((_BLK, _H), lambda i: (i, 0)),
            pl.BlockSpec((1, _BLK, _NC), lambda i: (i, 0, 0)),
            pl.BlockSpec((_H, _H), lambda i: (0, 0)),
            pl.BlockSpec((1, _H), lambda i: (0, 0)),
        ],
        out_specs=pl.BlockSpec((_BLK, _H), lambda i: (i, 0)),
        out_shape=jax.ShapeDtypeStruct((_NP, _H), jnp.float32),
    )(x, degt, w, b)


def _tc_gconv(aggp, degt, w, b, *, relu, oscale):
    def body(aggp_ref, deg_ref, w_ref, b_ref, o_ref):
        dd = deg_ref[0]
        iisq = lax.rsqrt(jnp.clip(dd[:, 1], 1.0, None))
        agg = (aggp_ref[0] + aggp_ref[1]) * iisq[:, None]
        h = jnp.dot(agg, w_ref[...], preferred_element_type=jnp.float32) + b_ref[...]
        if relu:
            h = jnp.maximum(h, 0.0)
        if oscale:
            oisq = lax.rsqrt(jnp.clip(dd[:, 0], 1.0, None))
            h = h * oisq[:, None]
        o_ref[...] = h

    return pl.pallas_call(
        body,
        grid=(_NP // _BLK,),
        in_specs=[
            pl.BlockSpec((_NC, _BLK, _H), lambda i: (0, i, 0)),
            pl.BlockSpec((1, _BLK, _NC), lambda i: (i, 0, 0)),
            pl.BlockSpec((_H, _H), lambda i: (0, 0)),
            pl.BlockSpec((1, _H), lambda i: (0, 0)),
        ],
        out_specs=pl.BlockSpec((_BLK, _H), lambda i: (i, 0)),
        out_shape=jax.ShapeDtypeStruct((_NP, _H), jnp.float32),
    )(aggp, degt, w, b)


def _tc_head(featsum, f_scale, w3, b3, w4, b4, w5p, b5p):
    r = featsum.shape[0]

    def body(f_ref, w3_ref, b3_ref, w4_ref, b4_ref, w5_ref, b5_ref, o_ref):
        z = f_ref[...] * f_scale
        z = jnp.maximum(jnp.dot(z, w3_ref[...], preferred_element_type=jnp.float32) + b3_ref[...], 0.0)
        z = jnp.maximum(jnp.dot(z, w4_ref[...], preferred_element_type=jnp.float32) + b4_ref[...], 0.0)
        o_ref[...] = jnp.dot(z, w5_ref[...], preferred_element_type=jnp.float32) + b5_ref[...]

    full = lambda i: (0, 0)
    return pl.pallas_call(
        body,
        grid=(r // _BLK,),
        in_specs=[
            pl.BlockSpec((_BLK, _H), lambda i: (i, 0)),
            pl.BlockSpec((_H, _H), full),
            pl.BlockSpec((1, _H), full),
            pl.BlockSpec((_H, _H), full),
            pl.BlockSpec((1, _H), full),
            pl.BlockSpec((_H, _H), full),
            pl.BlockSpec((1, _H), full),
        ],
        out_specs=pl.BlockSpec((_BLK, _H), lambda i: (i, 0)),
        out_shape=jax.ShapeDtypeStruct((r, _H), jnp.float32),
    )(featsum, w3, b3, w4, b4, w5p, b5p)


def kernel(x, edge_index, row_node_idx, W_in, b_in, W1, b1, W2, b2, W3, b3, W4, b4, W5, b5):
    r, f = row_node_idx.shape
    epw = _E // _NW
    e_ch = 125
    e_nch = epw // e_ch
    src4 = edge_index[0].reshape(_NW, 2, e_nch // 2, e_ch)
    dst4 = edge_index[1].reshape(_NW, 2, e_nch // 2, e_ch)
    eidx = jnp.stack([src4, dst4], axis=2)

    epw_d = _E // _NS
    d_ch = 125
    didx = jnp.stack(
        [edge_index[0].reshape(_NS, epw_d // d_ch, d_ch),
         edge_index[1].reshape(_NS, epw_d // d_ch, d_ch)]
    ).reshape(_NW, epw_d // d_ch, d_ch)

    rows_per_core = r // _NC
    rf_pw = r * f // _NW
    r_ch = 128
    gidx_row = row_node_idx.reshape(_NW, 1, 1, rf_pw // r_ch, r_ch)
    sidx_row = (lax.broadcasted_iota(jnp.int32, (r, f), 0) % rows_per_core).reshape(
        _NW, 1, 1, rf_pw // r_ch, r_ch
    )
    ridx = jnp.concatenate([gidx_row, sidx_row], axis=2)

    ones128 = jnp.ones((d_ch, _H), jnp.float32)
    zrow = jnp.zeros((_NP // _NS, _H), jnp.float32)
    x_p = jnp.pad(x, ((0, _NP - _N), (0, 0)))

    dpo = _deg_kernel(epw_d // d_ch, d_ch)(didx, ones128, zrow)
    degt = dpo[:, 0].reshape(_NC, _NP).T.reshape(_NP // _BLK, _BLK, _NC)

    h0s = _tc_input_linear(x_p, degt, W_in, b_in.reshape(1, _H))

    gs_edge = _gs_kernel(e_nch, e_ch, _NP, 2)
    agg1 = gs_edge(h0s, eidx, zrow).reshape(_NC, _NP, _H)
    h1s = _tc_gconv(agg1, degt, W1, b1.reshape(1, _H), relu=True, oscale=True)
    agg2 = gs_edge(h1s, eidx, zrow).reshape(_NC, _NP, _H)
    h2 = _tc_gconv(agg2, degt, W2, b2.reshape(1, _H), relu=False, oscale=False)

    gs_row = _gs_kernel(rf_pw // r_ch, r_ch, rows_per_core, 1)
    featsum = gs_row(h2, ridx, zrow)

    c = W5.shape[1]
    w5p = jnp.pad(W5, ((0, 0), (0, _H - c)))
    b5p = jnp.pad(b5, (0, _H - c)).reshape(1, _H)
    out = _tc_head(
        featsum, 1.0 / f, W3, b3.reshape(1, _H), W4, b4.reshape(1, _H), w5p, b5p
    )
    return out[:, :c]
